# baseline (speedup 1.0000x reference)
import functools

import jax
import jax.numpy as jnp
from jax.experimental import pallas as pl
from jax.experimental.pallas import tpu as pltpu

_N = 131072
_DIM = 2
_E = 8
_H = 64
_BLK = 4096


def _moe_body(x_ref, wr_ref, br_ref, w1_ref, b1_ref, w2_ref, b2_ref,
              w3_ref, b3_ref, out_ref, logits_ref):
  x = x_ref[...]
  logits = (jnp.dot(x, wr_ref[...].T, preferred_element_type=jnp.float32)
            + br_ref[...])
  logits_ref[...] = logits
  best = jnp.argmax(logits, axis=-1)
  acc = jnp.zeros((x.shape[0], _DIM), dtype=jnp.float32)
  for eid in range(_E):
    h1 = jnp.maximum(
        jnp.dot(x, w1_ref[eid].T, preferred_element_type=jnp.float32)
        + b1_ref[eid], 0.0)
    h2 = jnp.maximum(
        jnp.dot(h1, w2_ref[eid].T, preferred_element_type=jnp.float32)
        + b2_ref[eid], 0.0)
    ye = (jnp.dot(h2, w3_ref[eid].T, preferred_element_type=jnp.float32)
          + b3_ref[eid])
    acc = jnp.where((best == eid)[:, None], ye, acc)
  out_ref[...] = acc


@jax.jit
def kernel(x, Wr, br, W1, b1, W2, b2, W3, b3):
  n = x.shape[0]
  grid = (n // _BLK,)
  full = lambda *s: pl.BlockSpec(s, lambda i: (0,) * len(s))
  out, logits = pl.pallas_call(
      _moe_body,
      grid=grid,
      in_specs=[
          pl.BlockSpec((_BLK, _DIM), lambda i: (i, 0)),
          full(_E, _DIM), full(_E),
          full(_E, _H, _DIM), full(_E, _H),
          full(_E, _H, _H), full(_E, _H),
          full(_E, _DIM, _H), full(_E, _DIM),
      ],
      out_specs=[
          pl.BlockSpec((_BLK, _DIM), lambda i: (i, 0)),
          pl.BlockSpec((_BLK, _E), lambda i: (i, 0)),
      ],
      out_shape=[
          jax.ShapeDtypeStruct((n, _DIM), jnp.float32),
          jax.ShapeDtypeStruct((n, _E), jnp.float32),
      ],
  )(x, Wr, br, W1, b1, W2, b2, W3, b3)
  return out, logits
